# baseline (speedup 1.0000x reference)
import jax
import jax.numpy as jnp
from jax.experimental import pallas as pl
from jax.experimental.pallas import tpu as pltpu

N = 10000
K = 32
D = 128
P = 8
B = 256

A = 400
NCHUNK = N // A
KD = K * D
W = 64
P_MAX = (B + W - 1) // W
BP = B + W


def _silu(x):
    return x * jax.nn.sigmoid(x)


def _phase_a_kernel(meta_ref, x_ref, kdr_ref, sinc_ref, batch_ref,
                    w1t_ref, w2t_ref, gamma_ref, beta_ref, kfilt_ref,
                    g_ref, c_ref, s_ref, sf_acc):
    i = pl.program_id(0)

    @pl.when(i == 0)
    def _init():
        sf_acc[...] = jnp.zeros_like(sf_acc)

    x = x_ref[...]
    h = _silu(jnp.dot(x, w1t_ref[...], preferred_element_type=jnp.float32))
    h = _silu(jnp.dot(h, w2t_ref[...], preferred_element_type=jnp.float32))
    xr = x + h
    mean = jnp.mean(xr, axis=-1, keepdims=True)
    var = jnp.mean((xr - mean) ** 2, axis=-1, keepdims=True)
    xr = (xr - mean) * jax.lax.rsqrt(var + 1e-5) * gamma_ref[...] + beta_ref[...]

    sinc = sinc_ref[...]
    kdr = kdr_ref[...]
    c = jnp.cos(kdr) * sinc
    s = jnp.sin(kdr) * sinc
    c_ref[...] = c.astype(jnp.bfloat16)
    s_ref[...] = s.astype(jnp.bfloat16)

    xrb = xr.astype(jnp.bfloat16)
    cb = c.astype(jnp.bfloat16)
    sb = s.astype(jnp.bfloat16)
    zcat = jnp.concatenate(
        [cb[:, k:k + 1] * xrb for k in range(K)]
        + [sb[:, k:k + 1] * xrb for k in range(K)], axis=1)

    bvec = batch_ref[0]
    base = meta_ref[0, i]
    last = meta_ref[1, i]

    def _scatter_pass(off):
        ot = (jax.lax.broadcasted_iota(jnp.int32, (W, A), 0) + off
              == bvec).astype(jnp.bfloat16)
        off = pl.multiple_of(off, 8)
        sf_acc[pl.ds(off, W), :] += jax.lax.dot(
            ot, zcat, preferred_element_type=jnp.float32)

    _scatter_pass(base)
    for p in range(1, P_MAX):
        @pl.when(last >= base + p * W)
        def _overflow(p=p):
            _scatter_pass(base + p * W)

    @pl.when(i == NCHUNK - 1)
    def _emit():
        g_ref[...] = (kfilt_ref[...] * sf_acc[...]).astype(jnp.bfloat16)


def _phase_b_kernel(meta_ref, x_ref, batch_ref, c_ref, s_ref, g_ref,
                    wu1t_ref, wu2t_ref, out_ref):
    i = pl.program_id(0)
    bvec = batch_ref[0]
    base = meta_ref[0, i]
    last = meta_ref[1, i]
    c = c_ref[...].astype(jnp.float32)
    s = s_ref[...].astype(jnp.float32)

    def _gather_pass(off):
        ot = (jax.lax.broadcasted_iota(jnp.int32, (W, A), 0) + off
              == bvec).astype(jnp.bfloat16)
        gcat = jax.lax.dot_general(
            ot, g_ref[pl.ds(pl.multiple_of(off, 8), W), :],
            (((0,), (0,)), ((), ())),
            preferred_element_type=jnp.float32)
        ew = jnp.zeros((A, D), dtype=jnp.float32)
        for k in range(K):
            ew += c[:, k:k + 1] * gcat[:, k * D:(k + 1) * D]
            ew += s[:, k:k + 1] * gcat[:, KD + k * D:KD + (k + 1) * D]
        return ew

    ew_acc = _gather_pass(base)
    for p in range(1, P_MAX):
        ew_acc = jax.lax.cond(last >= base + p * W,
                              lambda ew: ew + _gather_pass(base + p * W),
                              lambda ew: ew, ew_acc)

    x_new = x_ref[...] + ew_acc
    u = _silu(jnp.dot(x_new, wu1t_ref[...], preferred_element_type=jnp.float32))
    u = _silu(jnp.dot(u, wu2t_ref[...], preferred_element_type=jnp.float32))
    out_ref[...] = x_new + u


@jax.jit
def kernel(x_scalar, k_dot_r, sinc_damping, batch, down_projection,
           W_pre1, W_pre2, gamma, beta, W_up, W_upd1, W_upd2):
    batch3 = batch.reshape(NCHUNK, 1, A)
    base = (batch[::A] // 8) * 8
    last = batch[A - 1::A]
    meta = jnp.stack([base, last]).astype(jnp.int32)
    kf = (down_projection @ W_up.T).reshape(1, KD)
    kfilt = jnp.concatenate([kf, kf], axis=1)
    gamma2 = gamma.reshape(1, D)
    beta2 = beta.reshape(1, D)

    chunk = lambda i, m: (i, 0)
    whole = lambda i, m: (0, 0)

    g, c_all, s_all = pl.pallas_call(
        _phase_a_kernel,
        grid_spec=pltpu.PrefetchScalarGridSpec(
            num_scalar_prefetch=1,
            grid=(NCHUNK,),
            in_specs=[
                pl.BlockSpec((A, D), chunk),
                pl.BlockSpec((A, K), chunk),
                pl.BlockSpec((A, K), chunk),
                pl.BlockSpec((1, 1, A), lambda i, m: (i, 0, 0)),
                pl.BlockSpec((D, D), whole),
                pl.BlockSpec((D, D), whole),
                pl.BlockSpec((1, D), whole),
                pl.BlockSpec((1, D), whole),
                pl.BlockSpec((1, 2 * KD), whole),
            ],
            out_specs=[
                pl.BlockSpec((BP, 2 * KD), whole),
                pl.BlockSpec((A, K), chunk),
                pl.BlockSpec((A, K), chunk),
            ],
            scratch_shapes=[pltpu.VMEM((BP, 2 * KD), jnp.float32)],
        ),
        out_shape=[
            jax.ShapeDtypeStruct((BP, 2 * KD), jnp.bfloat16),
            jax.ShapeDtypeStruct((N, K), jnp.bfloat16),
            jax.ShapeDtypeStruct((N, K), jnp.bfloat16),
        ],
    )(meta, x_scalar, k_dot_r, sinc_damping, batch3,
      W_pre1.T, W_pre2.T, gamma2, beta2, kfilt)

    out = pl.pallas_call(
        _phase_b_kernel,
        grid_spec=pltpu.PrefetchScalarGridSpec(
            num_scalar_prefetch=1,
            grid=(NCHUNK,),
            in_specs=[
                pl.BlockSpec((A, D), chunk),
                pl.BlockSpec((1, 1, A), lambda i, m: (i, 0, 0)),
                pl.BlockSpec((A, K), chunk),
                pl.BlockSpec((A, K), chunk),
                pl.BlockSpec((BP, 2 * KD), whole),
                pl.BlockSpec((D, D), whole),
                pl.BlockSpec((D, D), whole),
            ],
            out_specs=pl.BlockSpec((A, D), chunk),
        ),
        out_shape=jax.ShapeDtypeStruct((N, D), jnp.float32),
    )(meta, x_scalar, batch3, c_all, s_all, g, W_upd1.T, W_upd2.T)

    return out


# speedup vs baseline: 1.1485x; 1.1485x over previous
import jax
import jax.numpy as jnp
from jax.experimental import pallas as pl
from jax.experimental.pallas import tpu as pltpu

N = 10000
K = 32
D = 128
P = 8
B = 256

A = 1000
NCHUNK = N // A
KD = K * D
W = 64
J = 2 * K
BP = B + W


def _silu(x):
    return x * jax.nn.sigmoid(x)


def _phase_a_kernel(meta_ref, x_ref, kdrt_ref, sinct_ref, batch_ref,
                    w1t_ref, w2t_ref, gamma_ref, beta_ref, kf2_ref,
                    g_ref, sf_acc):
    i = pl.program_id(0)

    @pl.when(i == 0)
    def _init():
        sf_acc[...] = jnp.zeros_like(sf_acc)

    x = x_ref[...]
    h = _silu(jnp.dot(x, w1t_ref[...], preferred_element_type=jnp.float32))
    h = _silu(jnp.dot(h, w2t_ref[...], preferred_element_type=jnp.float32))
    xr = x + h
    mean = jnp.mean(xr, axis=-1, keepdims=True)
    var = jnp.mean((xr - mean) ** 2, axis=-1, keepdims=True)
    xr = (xr - mean) * jax.lax.rsqrt(var + 1e-5) * gamma_ref[...] + beta_ref[...]
    xrb = xr.astype(jnp.bfloat16)

    sinct = sinct_ref[0]
    kdrt = kdrt_ref[0]
    ct = (jnp.cos(kdrt) * sinct).astype(jnp.bfloat16)
    st = (jnp.sin(kdrt) * sinct).astype(jnp.bfloat16)

    bvec = batch_ref[0]
    base = meta_ref[0, i]
    last = meta_ref[1, i]

    def _scatter_pass(off):
        ot = (jax.lax.broadcasted_iota(jnp.int32, (W, A), 0) + off
              == bvec).astype(jnp.bfloat16)
        mt = jnp.concatenate(
            [ot * ct[k:k + 1, :] for k in range(K)]
            + [ot * st[k:k + 1, :] for k in range(K)], axis=0)
        res = jnp.dot(mt, xrb, preferred_element_type=jnp.float32)
        off8 = pl.multiple_of(off, 8)
        for j in range(J):
            sf_acc[pl.ds(j * BP + off8, W), :] += res[j * W:(j + 1) * W, :]

    _scatter_pass(base)

    def _body(p, carry):
        _scatter_pass(base + p * W)
        return carry

    npass = (last - base) // W + 1
    jax.lax.fori_loop(1, npass, _body, jnp.int32(0))

    @pl.when(i == NCHUNK - 1)
    def _emit():
        kf2 = kf2_ref[...]
        for j in range(J):
            g_ref[j * BP:(j + 1) * BP, :] = (
                kf2[j:j + 1, :] * sf_acc[j * BP:(j + 1) * BP, :]
            ).astype(jnp.bfloat16)


def _phase_b_kernel(meta_ref, x_ref, kdr_ref, sinc_ref, bcol_ref, g_ref,
                    wu1t_ref, wu2t_ref, out_ref):
    i = pl.program_id(0)
    base = meta_ref[0, i]
    last = meta_ref[1, i]
    sinc = sinc_ref[...]
    kdr = kdr_ref[...]
    cb = (jnp.cos(kdr) * sinc).astype(jnp.bfloat16)
    sb = (jnp.sin(kdr) * sinc).astype(jnp.bfloat16)
    bcol = bcol_ref[0]

    def _gather_pass(off):
        o = (jax.lax.broadcasted_iota(jnp.int32, (A, W), 1) + off
             == bcol).astype(jnp.bfloat16)
        m = jnp.concatenate(
            [cb[:, k:k + 1] * o for k in range(K)]
            + [sb[:, k:k + 1] * o for k in range(K)], axis=1)
        off8 = pl.multiple_of(off, 8)
        gwin = jnp.concatenate(
            [g_ref[pl.ds(j * BP + off8, W), :] for j in range(J)],
            axis=0)
        return jnp.dot(m, gwin, preferred_element_type=jnp.float32)

    ew = _gather_pass(base)

    def _body(p, acc):
        return acc + _gather_pass(base + p * W)

    npass = (last - base) // W + 1
    ew = jax.lax.fori_loop(1, npass, _body, ew)

    x_new = x_ref[...] + ew
    u = _silu(jnp.dot(x_new, wu1t_ref[...], preferred_element_type=jnp.float32))
    u = _silu(jnp.dot(u, wu2t_ref[...], preferred_element_type=jnp.float32))
    out_ref[...] = x_new + u


@jax.jit
def kernel(x_scalar, k_dot_r, sinc_damping, batch, down_projection,
           W_pre1, W_pre2, gamma, beta, W_up, W_upd1, W_upd2):
    batch_row = batch.reshape(NCHUNK, 1, A)
    batch_col = batch.reshape(NCHUNK, A, 1)
    base = (batch[::A] // 8) * 8
    last = batch[A - 1::A]
    meta = jnp.stack([base, last]).astype(jnp.int32)
    kf = down_projection @ W_up.T
    kf2 = jnp.concatenate([kf, kf], axis=0)
    gamma2 = gamma.reshape(1, D)
    beta2 = beta.reshape(1, D)

    chunk = lambda i, m: (i, 0)
    whole = lambda i, m: (0, 0)

    g = pl.pallas_call(
        _phase_a_kernel,
        grid_spec=pltpu.PrefetchScalarGridSpec(
            num_scalar_prefetch=1,
            grid=(NCHUNK,),
            in_specs=[
                pl.BlockSpec((A, D), chunk),
                pl.BlockSpec((1, K, A), lambda i, m: (i, 0, 0)),
                pl.BlockSpec((1, K, A), lambda i, m: (i, 0, 0)),
                pl.BlockSpec((1, 1, A), lambda i, m: (i, 0, 0)),
                pl.BlockSpec((D, D), whole),
                pl.BlockSpec((D, D), whole),
                pl.BlockSpec((1, D), whole),
                pl.BlockSpec((1, D), whole),
                pl.BlockSpec((J, D), whole),
            ],
            out_specs=pl.BlockSpec((J * BP, D), whole),
            scratch_shapes=[pltpu.VMEM((J * BP, D), jnp.float32)],
        ),
        out_shape=jax.ShapeDtypeStruct((J * BP, D), jnp.bfloat16),
    )(meta, x_scalar,
      k_dot_r.T.reshape(K, NCHUNK, A).transpose(1, 0, 2),
      sinc_damping.T.reshape(K, NCHUNK, A).transpose(1, 0, 2),
      batch_row, W_pre1.T, W_pre2.T, gamma2, beta2, kf2)

    out = pl.pallas_call(
        _phase_b_kernel,
        grid_spec=pltpu.PrefetchScalarGridSpec(
            num_scalar_prefetch=1,
            grid=(NCHUNK,),
            in_specs=[
                pl.BlockSpec((A, D), chunk),
                pl.BlockSpec((A, K), chunk),
                pl.BlockSpec((A, K), chunk),
                pl.BlockSpec((1, A, 1), lambda i, m: (i, 0, 0)),
                pl.BlockSpec((J * BP, D), whole),
                pl.BlockSpec((D, D), whole),
                pl.BlockSpec((D, D), whole),
            ],
            out_specs=pl.BlockSpec((A, D), chunk),
        ),
        out_shape=jax.ShapeDtypeStruct((N, D), jnp.float32),
    )(meta, x_scalar, k_dot_r, sinc_damping, batch_col,
      g, W_upd1.T, W_upd2.T)

    return out


# speedup vs baseline: 1.3158x; 1.1457x over previous
import jax
import jax.numpy as jnp
from jax.experimental import pallas as pl
from jax.experimental.pallas import tpu as pltpu

N = 10000
K = 32
D = 128
P = 8
B = 256

A = 1000
NCHUNK = N // A
KD = K * D
W = 64
J = 2 * K
BP = B + W


def _silu(x):
    return x * jax.nn.sigmoid(x)


def _phase_a_kernel(meta_ref, x_ref, kdrt_ref, sinct_ref, batch_ref,
                    w1t_ref, w2t_ref, gamma_ref, beta_ref, kf2_ref,
                    g_ref, sf_acc):
    i = pl.program_id(0)

    @pl.when(i == 0)
    def _init():
        sf_acc[...] = jnp.zeros_like(sf_acc)

    x = x_ref[...]
    h = _silu(jnp.dot(x, w1t_ref[...], preferred_element_type=jnp.float32))
    h = _silu(jnp.dot(h, w2t_ref[...], preferred_element_type=jnp.float32))
    xr = x + h
    mean = jnp.mean(xr, axis=-1, keepdims=True)
    var = jnp.mean((xr - mean) ** 2, axis=-1, keepdims=True)
    xr = (xr - mean) * jax.lax.rsqrt(var + 1e-5) * gamma_ref[...] + beta_ref[...]
    xrb = xr.astype(jnp.bfloat16)

    sinct = sinct_ref[0]
    kdrt = kdrt_ref[0]
    ct = (jnp.cos(kdrt) * sinct).astype(jnp.bfloat16)
    st = (jnp.sin(kdrt) * sinct).astype(jnp.bfloat16)

    bvec = batch_ref[0]
    base = meta_ref[0, i]
    last = meta_ref[1, i]

    def _scatter_pass(off):
        ot = (jax.lax.broadcasted_iota(jnp.int32, (W, A), 0) + off
              == bvec).astype(jnp.bfloat16)
        pieces = []
        for k in range(K):
            pieces.append(ot * ct[k:k + 1, :])
            pieces.append(ot * st[k:k + 1, :])
        mt = jnp.concatenate(pieces, axis=0)
        res = jnp.dot(mt, xrb, preferred_element_type=jnp.float32)
        off8 = pl.multiple_of(off, 8)
        for j in range(J):
            sf_acc[pl.ds(j * BP + off8, W), :] += res[j * W:(j + 1) * W, :]

    _scatter_pass(base)

    def _body(p, carry):
        _scatter_pass(base + p * W)
        return carry

    npass = (last - base) // W + 1
    jax.lax.fori_loop(1, npass, _body, jnp.int32(0))

    @pl.when(i == NCHUNK - 1)
    def _emit():
        kf2 = kf2_ref[...]
        for j in range(J):
            g_ref[j * BP:(j + 1) * BP, :] = (
                kf2[j:j + 1, :] * sf_acc[j * BP:(j + 1) * BP, :]
            ).astype(jnp.bfloat16)


def _phase_b_kernel(meta_ref, x_ref, kdr_ref, sinc_ref, bcol_ref, g_ref,
                    wu1t_ref, wu2t_ref, out_ref):
    i = pl.program_id(0)
    base = meta_ref[0, i]
    last = meta_ref[1, i]
    sinc = sinc_ref[...]
    kdr = kdr_ref[...]
    cb = (jnp.cos(kdr) * sinc).astype(jnp.bfloat16)
    sb = (jnp.sin(kdr) * sinc).astype(jnp.bfloat16)
    bcol = bcol_ref[0]

    lane = jax.lax.broadcasted_iota(jnp.int32, (A, 2 * W), 1)
    cself = (lane < W).astype(jnp.bfloat16)
    nself = jnp.bfloat16(1) - cself

    def _gather_pass(off):
        o2 = ((lane & (W - 1)) + off == bcol).astype(jnp.bfloat16)
        o2c = o2 * cself
        o2s = o2 * nself
        m = jnp.concatenate(
            [o2c * cb[:, k:k + 1] + o2s * sb[:, k:k + 1]
             for k in range(K)], axis=1)
        off8 = pl.multiple_of(off, 8)
        gwin = jnp.concatenate(
            [g_ref[pl.ds(j * BP + off8, W), :] for j in range(J)],
            axis=0)
        return jnp.dot(m, gwin, preferred_element_type=jnp.float32)

    ew = _gather_pass(base)

    def _body(p, acc):
        return acc + _gather_pass(base + p * W)

    npass = (last - base) // W + 1
    ew = jax.lax.fori_loop(1, npass, _body, ew)

    x_new = x_ref[...] + ew
    u = _silu(jnp.dot(x_new, wu1t_ref[...], preferred_element_type=jnp.float32))
    u = _silu(jnp.dot(u, wu2t_ref[...], preferred_element_type=jnp.float32))
    out_ref[...] = x_new + u


@jax.jit
def kernel(x_scalar, k_dot_r, sinc_damping, batch, down_projection,
           W_pre1, W_pre2, gamma, beta, W_up, W_upd1, W_upd2):
    batch_row = batch.reshape(NCHUNK, 1, A)
    batch_col = batch.reshape(NCHUNK, A, 1)
    base = (batch[::A] // 8) * 8
    last = batch[A - 1::A]
    meta = jnp.stack([base, last]).astype(jnp.int32)
    kf = down_projection @ W_up.T
    kf2 = jnp.repeat(kf, 2, axis=0)
    gamma2 = gamma.reshape(1, D)
    beta2 = beta.reshape(1, D)

    chunk = lambda i, m: (i, 0)
    whole = lambda i, m: (0, 0)

    g = pl.pallas_call(
        _phase_a_kernel,
        grid_spec=pltpu.PrefetchScalarGridSpec(
            num_scalar_prefetch=1,
            grid=(NCHUNK,),
            in_specs=[
                pl.BlockSpec((A, D), chunk),
                pl.BlockSpec((1, K, A), lambda i, m: (i, 0, 0)),
                pl.BlockSpec((1, K, A), lambda i, m: (i, 0, 0)),
                pl.BlockSpec((1, 1, A), lambda i, m: (i, 0, 0)),
                pl.BlockSpec((D, D), whole),
                pl.BlockSpec((D, D), whole),
                pl.BlockSpec((1, D), whole),
                pl.BlockSpec((1, D), whole),
                pl.BlockSpec((J, D), whole),
            ],
            out_specs=pl.BlockSpec((J * BP, D), whole),
            scratch_shapes=[pltpu.VMEM((J * BP, D), jnp.float32)],
        ),
        out_shape=jax.ShapeDtypeStruct((J * BP, D), jnp.bfloat16),
    )(meta, x_scalar,
      k_dot_r.T.reshape(K, NCHUNK, A).transpose(1, 0, 2),
      sinc_damping.T.reshape(K, NCHUNK, A).transpose(1, 0, 2),
      batch_row, W_pre1.T, W_pre2.T, gamma2, beta2, kf2)

    out = pl.pallas_call(
        _phase_b_kernel,
        grid_spec=pltpu.PrefetchScalarGridSpec(
            num_scalar_prefetch=1,
            grid=(NCHUNK,),
            in_specs=[
                pl.BlockSpec((A, D), chunk),
                pl.BlockSpec((A, K), chunk),
                pl.BlockSpec((A, K), chunk),
                pl.BlockSpec((1, A, 1), lambda i, m: (i, 0, 0)),
                pl.BlockSpec((J * BP, D), whole),
                pl.BlockSpec((D, D), whole),
                pl.BlockSpec((D, D), whole),
            ],
            out_specs=pl.BlockSpec((A, D), chunk),
        ),
        out_shape=jax.ShapeDtypeStruct((N, D), jnp.float32),
    )(meta, x_scalar, k_dot_r, sinc_damping, batch_col,
      g, W_upd1.T, W_upd2.T)

    return out


# speedup vs baseline: 1.3477x; 1.0242x over previous
import jax
import jax.numpy as jnp
from jax.experimental import pallas as pl
from jax.experimental.pallas import tpu as pltpu

N = 10000
K = 32
D = 128
P = 8
B = 256

A = 1000
NCHUNK = N // A
KD = K * D
W = 64
J = 2 * K
BP = B + W


def _silu(x):
    return x * jax.nn.sigmoid(x)


def _phase_a_kernel(meta_ref, x_ref, kdrt_ref, sinct_ref, batch_ref,
                    w1t_ref, w2t_ref, gamma_ref, beta_ref, kf3_ref, e1t_ref,
                    g_ref, cbsb_ref, sf_acc):
    i = pl.program_id(0)

    @pl.when(i == 0)
    def _init():
        sf_acc[...] = jnp.zeros_like(sf_acc)

    x = x_ref[...]
    h = _silu(jnp.dot(x, w1t_ref[...], preferred_element_type=jnp.float32))
    h = _silu(jnp.dot(h, w2t_ref[...], preferred_element_type=jnp.float32))
    xr = x + h
    mean = jnp.mean(xr, axis=-1, keepdims=True)
    var = jnp.mean((xr - mean) ** 2, axis=-1, keepdims=True)
    xr = (xr - mean) * jax.lax.rsqrt(var + 1e-5) * gamma_ref[...] + beta_ref[...]
    xrb = xr.astype(jnp.bfloat16)

    sinct = sinct_ref[0]
    kdrt = kdrt_ref[0]
    ct = (jnp.cos(kdrt) * sinct).astype(jnp.bfloat16)
    st = (jnp.sin(kdrt) * sinct).astype(jnp.bfloat16)
    ctst = jnp.concatenate([ct, st], axis=0)
    cbsb_ref[...] = ctst.T

    coef_exp = jnp.dot(e1t_ref[...], ctst,
                       preferred_element_type=jnp.float32
                       ).astype(jnp.bfloat16)

    bvec = batch_ref[0]
    base = meta_ref[0, i]
    last = meta_ref[1, i]

    def _scatter_pass(off):
        ot = (jax.lax.broadcasted_iota(jnp.int32, (W, A), 0) + off
              == bvec).astype(jnp.bfloat16)
        mt = coef_exp * jnp.concatenate([ot] * J, axis=0)
        res = jnp.dot(mt, xrb, preferred_element_type=jnp.float32)
        off8 = pl.multiple_of(off, 8)
        sf_acc[:, pl.ds(off8, W), :] += res.reshape(J, W, D)

    _scatter_pass(base)

    def _body(p, carry):
        _scatter_pass(base + p * W)
        return carry

    npass = (last - base) // W + 1
    jax.lax.fori_loop(1, npass, _body, jnp.int32(0))

    @pl.when(i == NCHUNK - 1)
    def _emit():
        g_ref[...] = (kf3_ref[...] * sf_acc[...]).astype(jnp.bfloat16)


def _phase_b_kernel(meta_ref, x_ref, cbsb_ref, bcol_ref, g_ref, e1b_ref,
                    wu1t_ref, wu2t_ref, out_ref):
    i = pl.program_id(0)
    base = meta_ref[0, i]
    last = meta_ref[1, i]
    cbsb = cbsb_ref[...]
    bcol = bcol_ref[0]

    coef_exp = jnp.dot(cbsb, e1b_ref[...],
                       preferred_element_type=jnp.float32
                       ).astype(jnp.bfloat16)
    lane = jax.lax.broadcasted_iota(jnp.int32, (A, 2 * W), 1)

    def _gather_pass(off):
        o2 = ((lane & (W - 1)) + off == bcol).astype(jnp.bfloat16)
        m = coef_exp * jnp.concatenate([o2] * K, axis=1)
        off8 = pl.multiple_of(off, 8)
        gwin = g_ref[:, pl.ds(off8, W), :].reshape(J * W, D)
        return jnp.dot(m, gwin, preferred_element_type=jnp.float32)

    ew = _gather_pass(base)

    def _body(p, acc):
        return acc + _gather_pass(base + p * W)

    npass = (last - base) // W + 1
    ew = jax.lax.fori_loop(1, npass, _body, ew)

    x_new = x_ref[...] + ew
    u = _silu(jnp.dot(x_new, wu1t_ref[...], preferred_element_type=jnp.float32))
    u = _silu(jnp.dot(u, wu2t_ref[...], preferred_element_type=jnp.float32))
    out_ref[...] = x_new + u


@jax.jit
def kernel(x_scalar, k_dot_r, sinc_damping, batch, down_projection,
           W_pre1, W_pre2, gamma, beta, W_up, W_upd1, W_upd2):
    batch_row = batch.reshape(NCHUNK, 1, A)
    batch_col = batch.reshape(NCHUNK, A, 1)
    base = (batch[::A] // 8) * 8
    last = batch[A - 1::A]
    meta = jnp.stack([base, last]).astype(jnp.int32)
    kf = down_projection @ W_up.T
    kf3 = jnp.repeat(kf, 2, axis=0).reshape(J, 1, D)
    gamma2 = gamma.reshape(1, D)
    beta2 = beta.reshape(1, D)

    q_of_row = jnp.arange(J * W, dtype=jnp.int32) // W
    src_row = (q_of_row // 2) + (q_of_row % 2) * K
    e1t = (src_row[:, None] == jnp.arange(J, dtype=jnp.int32)[None, :]
           ).astype(jnp.bfloat16)
    col = jnp.arange(J * W, dtype=jnp.int32)
    src_col = (col // (2 * W)) + ((col // W) % 2) * K
    e1b = (jnp.arange(J, dtype=jnp.int32)[:, None] == src_col[None, :]
           ).astype(jnp.bfloat16)

    chunk = lambda i, m: (i, 0)
    whole = lambda i, m: (0, 0)

    g, cbsb_all = pl.pallas_call(
        _phase_a_kernel,
        grid_spec=pltpu.PrefetchScalarGridSpec(
            num_scalar_prefetch=1,
            grid=(NCHUNK,),
            in_specs=[
                pl.BlockSpec((A, D), chunk),
                pl.BlockSpec((1, K, A), lambda i, m: (i, 0, 0)),
                pl.BlockSpec((1, K, A), lambda i, m: (i, 0, 0)),
                pl.BlockSpec((1, 1, A), lambda i, m: (i, 0, 0)),
                pl.BlockSpec((D, D), whole),
                pl.BlockSpec((D, D), whole),
                pl.BlockSpec((1, D), whole),
                pl.BlockSpec((1, D), whole),
                pl.BlockSpec((J, 1, D), lambda i, m: (0, 0, 0)),
                pl.BlockSpec((J * W, J), whole),
            ],
            out_specs=[
                pl.BlockSpec((J, BP, D), lambda i, m: (0, 0, 0)),
                pl.BlockSpec((A, J), chunk),
            ],
            scratch_shapes=[pltpu.VMEM((J, BP, D), jnp.float32)],
        ),
        out_shape=[
            jax.ShapeDtypeStruct((J, BP, D), jnp.bfloat16),
            jax.ShapeDtypeStruct((N, J), jnp.bfloat16),
        ],
    )(meta, x_scalar,
      k_dot_r.T.reshape(K, NCHUNK, A).transpose(1, 0, 2),
      sinc_damping.T.reshape(K, NCHUNK, A).transpose(1, 0, 2),
      batch_row, W_pre1.T, W_pre2.T, gamma2, beta2, kf3, e1t)

    out = pl.pallas_call(
        _phase_b_kernel,
        grid_spec=pltpu.PrefetchScalarGridSpec(
            num_scalar_prefetch=1,
            grid=(NCHUNK,),
            in_specs=[
                pl.BlockSpec((A, D), chunk),
                pl.BlockSpec((A, J), chunk),
                pl.BlockSpec((1, A, 1), lambda i, m: (i, 0, 0)),
                pl.BlockSpec((J, BP, D), lambda i, m: (0, 0, 0)),
                pl.BlockSpec((J, J * W), whole),
                pl.BlockSpec((D, D), whole),
                pl.BlockSpec((D, D), whole),
            ],
            out_specs=pl.BlockSpec((A, D), chunk),
        ),
        out_shape=jax.ShapeDtypeStruct((N, D), jnp.float32),
    )(meta, x_scalar, cbsb_all, batch_col, g, e1b, W_upd1.T, W_upd2.T)

    return out
